# baseline (speedup 1.0000x reference)
import jax
import jax.numpy as jnp
from jax.experimental import pallas as pl
from jax.experimental.pallas import tpu as pltpu

_TOPK = 8
_NE = 64
_BLOCK = 512
_CHUNK = 64


def _gate_kernel(x_ref, w_ref, b_ref, wts_ref, idx_ref):
    x = x_ref[...]
    w = w_ref[...]
    logits = jax.lax.dot_general(
        x, w, (((1,), (1,)), ((), ())), preferred_element_type=jnp.float32
    )
    logits = logits + b_ref[...]

    neg_inf = jnp.float32(-jnp.inf)
    for c in range(_BLOCK // _CHUNK):
        cur = logits[c * _CHUNK:(c + 1) * _CHUNK, :]
        colsf = jax.lax.broadcasted_iota(jnp.int32, cur.shape, 1).astype(
            jnp.float32
        )
        nef = jnp.float32(_NE)
        vals = []
        idxs = []
        for _ in range(_TOPK):
            m = jnp.max(cur, axis=-1, keepdims=True)
            idxf = jnp.min(
                jnp.where(cur == m, colsf, nef), axis=-1, keepdims=True
            )
            vals.append(m)
            idxs.append(idxf)
            cur = jnp.where(colsf == idxf, neg_inf, cur)

        top = jnp.concatenate(vals, axis=-1)
        tidx = jnp.concatenate(idxs, axis=-1).astype(jnp.int32)
        e = jnp.exp(top - top[:, 0:1])
        wts_ref[c * _CHUNK:(c + 1) * _CHUNK, :] = e / jnp.sum(
            e, axis=-1, keepdims=True
        )
        idx_ref[c * _CHUNK:(c + 1) * _CHUNK, :] = tidx


def kernel(x, W, b):
    n, d = x.shape
    grid = (n // _BLOCK,)
    wts, idx = pl.pallas_call(
        _gate_kernel,
        grid=grid,
        in_specs=[
            pl.BlockSpec((_BLOCK, d), lambda i: (i, 0)),
            pl.BlockSpec((_NE, d), lambda i: (0, 0)),
            pl.BlockSpec((1, _NE), lambda i: (0, 0)),
        ],
        out_specs=[
            pl.BlockSpec((_BLOCK, _TOPK), lambda i: (i, 0)),
            pl.BlockSpec((_BLOCK, _TOPK), lambda i: (i, 0)),
        ],
        out_shape=[
            jax.ShapeDtypeStruct((n, _TOPK), jnp.float32),
            jax.ShapeDtypeStruct((n, _TOPK), jnp.int32),
        ],
        compiler_params=pltpu.CompilerParams(
            dimension_semantics=("parallel",),
        ),
    )(x, W, b.reshape(1, _NE))
    return wts, idx


# speedup vs baseline: 1.1809x; 1.1809x over previous
import jax
import jax.numpy as jnp
from jax.experimental import pallas as pl
from jax.experimental.pallas import tpu as pltpu

_TOPK = 8
_NE = 64
_BLOCK = 512
_CHUNK = 64


def _topk_tail(logits, wts_ref, idx_ref):
    neg_inf = jnp.float32(-jnp.inf)
    for c in range(_BLOCK // _CHUNK):
        cur = logits[c * _CHUNK:(c + 1) * _CHUNK, :]
        colsf = jax.lax.broadcasted_iota(jnp.int32, cur.shape, 1).astype(
            jnp.float32
        )
        nef = jnp.float32(_NE)
        vals = []
        idxs = []
        for _ in range(_TOPK):
            m = jnp.max(cur, axis=-1, keepdims=True)
            idxf = jnp.min(
                jnp.where(cur == m, colsf, nef), axis=-1, keepdims=True
            )
            vals.append(m)
            idxs.append(idxf)
            cur = jnp.where(colsf == idxf, neg_inf, cur)

        top = jnp.concatenate(vals, axis=-1)
        tidx = jnp.concatenate(idxs, axis=-1).astype(jnp.int32)
        e = jnp.exp(top - top[:, 0:1])
        wts_ref[c * _CHUNK:(c + 1) * _CHUNK, :] = e / jnp.sum(
            e, axis=-1, keepdims=True
        )
        idx_ref[c * _CHUNK:(c + 1) * _CHUNK, :] = tidx


def _gate_kernel(x_ref, w_ref, b_ref, wts_ref, idx_ref, lg_ref):
    i = pl.program_id(0)

    _topk_tail(lg_ref[jax.lax.rem(i + 1, 2)], wts_ref, idx_ref)

    x = x_ref[...]
    w = w_ref[...]
    lg_ref[jax.lax.rem(i, 2)] = (
        jax.lax.dot_general(
            x, w, (((1,), (1,)), ((), ())),
            preferred_element_type=jnp.float32,
        )
        + b_ref[...]
    )


def kernel(x, W, b):
    n, d = x.shape
    nb = n // _BLOCK
    wts, idx = pl.pallas_call(
        _gate_kernel,
        grid=(nb + 1,),
        in_specs=[
            pl.BlockSpec((_BLOCK, d), lambda i: (jnp.minimum(i, nb - 1), 0)),
            pl.BlockSpec((_NE, d), lambda i: (0, 0)),
            pl.BlockSpec((1, _NE), lambda i: (0, 0)),
        ],
        out_specs=[
            pl.BlockSpec(
                (_BLOCK, _TOPK), lambda i: (jnp.maximum(i - 1, 0), 0)
            ),
            pl.BlockSpec(
                (_BLOCK, _TOPK), lambda i: (jnp.maximum(i - 1, 0), 0)
            ),
        ],
        out_shape=[
            jax.ShapeDtypeStruct((n, _TOPK), jnp.float32),
            jax.ShapeDtypeStruct((n, _TOPK), jnp.int32),
        ],
        scratch_shapes=[pltpu.VMEM((2, _BLOCK, _NE), jnp.float32)],
        compiler_params=pltpu.CompilerParams(
            dimension_semantics=("arbitrary",),
        ),
    )(x, W, b.reshape(1, _NE))
    return wts, idx
